# Initial kernel scaffold; baseline (speedup 1.0000x reference)
import jax
import jax.numpy as jnp
from jax.experimental import pallas as pl


def kernel(nodes, edges, W1, b1, Wg1, bg1, Wg2, bg2, Wout, bout, Wc1, bc1, Wc2, bc2, Wc3, bc3, Wc4, bc4):
    raise NotImplementedError("write your pallas kernel here")


# speedup vs baseline: 4.1340x; 4.1340x over previous
import functools

import jax
import jax.numpy as jnp
from jax import lax
from jax.experimental import pallas as pl
from jax.experimental.pallas import tpu as pltpu
from jax.experimental.pallas import tpu_sc as plsc

N = 50000
E = 1600000
NT = 50048
DUMMY = 50040
EP_ROWS = 12544
HID = 256
SL = 16
NSL = HID // SL
NC = 2
NS = 16
CH = 8
STRIPE = NT // NS
DSTRIPE = NT // 8

_MESH = plsc.VectorSubcoreMesh(
    core_axis_name="c", subcore_axis_name="s", num_cores=NC, num_subcores=NS)
_SC_PARAMS = pltpu.CompilerParams(use_tc_tiling_on_sc=False)


def _deg_body(src2, dst2, zdeg, out, src_buf, dst_buf, ones, bnc,
              h_out, h_in):
  c = lax.axis_index("c")
  s = lax.axis_index("s")
  w = c * NS + s
  for i in range(8):
    ones[pl.ds(i * 16, 16)] = jnp.full((16,), 1.0, jnp.float32)
  pltpu.sync_copy(zdeg, bnc)

  @pl.when(s < 8)
  def _():
    pltpu.sync_copy(bnc, h_out.at[pl.ds(s * DSTRIPE, DSTRIPE)])

  @pl.when(s >= 8)
  def _():
    pltpu.sync_copy(bnc, h_in.at[pl.ds((s - 8) * DSTRIPE, DSTRIPE)])

  plsc.subcore_barrier()
  row0 = w * 392

  def step(i, carry):
    off = row0 + i * CH
    pltpu.sync_copy(src2.at[pl.ds(off, CH), :], src_buf)
    pltpu.sync_copy(dst2.at[pl.ds(off, CH), :], dst_buf)
    for j in range(CH):
      pltpu.sync_copy(ones, h_out.at[src_buf.at[j]], add=True)
      pltpu.sync_copy(ones, h_in.at[dst_buf.at[j]], add=True)
    return carry

  lax.fori_loop(0, 49, step, 0)
  plsc.subcore_barrier()
  for half, hist in ((0, h_out), (1, h_in)):
    base = c * 2 * NT + half * NT
    pltpu.sync_copy(hist.at[pl.ds(s * STRIPE, STRIPE)],
                    bnc.at[pl.ds(0, STRIPE)])
    pltpu.sync_copy(bnc.at[pl.ds(0, STRIPE)],
                    out.at[pl.ds(base + s * STRIPE, STRIPE)])


_sc_degrees = pl.kernel(
    _deg_body,
    out_type=jax.ShapeDtypeStruct((NC * 2 * NT,), jnp.float32),
    mesh=_MESH,
    compiler_params=_SC_PARAMS,
    scratch_types=[
        pltpu.VMEM((CH, 128), jnp.int32),
        pltpu.VMEM((CH, 128), jnp.int32),
        pltpu.VMEM((128,), jnp.float32),
        pltpu.VMEM((DSTRIPE,), jnp.float32),
        pltpu.VMEM_SHARED((NT,), jnp.float32),
        pltpu.VMEM_SHARED((NT,), jnp.float32),
    ],
)


def _agg1_body(x4, src2, dst2, z32, out,
               src_buf, dst_buf, idx_buf, rows, sem, bnc, accum):
  c = lax.axis_index("c")
  s = lax.axis_index("s")
  w = c * NS + s
  pltpu.sync_copy(z32, bnc)
  pltpu.sync_copy(bnc, accum.at[pl.ds(s * STRIPE, STRIPE), :])
  plsc.subcore_barrier()
  row0 = w * 392

  def step(i, carry):
    off = row0 + i * CH
    pltpu.sync_copy(src2.at[pl.ds(off, CH), :], src_buf)
    pltpu.sync_copy(dst2.at[pl.ds(off, CH), :], dst_buf)
    for j in range(CH):
      for v in range(8):
        idx_buf[pl.ds(v * 16, 16)] = src_buf[j, pl.ds(v * 16, 16)] * 8
      pltpu.async_copy(x4.at[idx_buf], rows, sem).wait()
      pltpu.sync_copy(rows, accum.at[dst_buf.at[j]], add=True)
    return carry

  lax.fori_loop(0, 49, step, 0)
  plsc.subcore_barrier()
  pltpu.sync_copy(accum.at[pl.ds(s * STRIPE, STRIPE), :], bnc)
  pltpu.sync_copy(bnc, out.at[c, pl.ds(s * STRIPE, STRIPE), pl.ds(0, SL)])


_sc_agg1 = pl.kernel(
    _agg1_body,
    out_type=jax.ShapeDtypeStruct((NC, NT, 128), jnp.float32),
    mesh=_MESH,
    compiler_params=_SC_PARAMS,
    scratch_types=[
        pltpu.VMEM((CH, 128), jnp.int32),
        pltpu.VMEM((CH, 128), jnp.int32),
        pltpu.VMEM((128,), jnp.int32),
        pltpu.VMEM((128, SL), jnp.float32),
        pltpu.SemaphoreType.DMA,
        pltpu.VMEM((STRIPE, SL), jnp.float32),
        pltpu.VMEM_SHARED((NT, SL), jnp.float32),
    ],
)


def _agg8_body(x8, src2, dst2, z32, out,
               src_buf, dst_buf, idx_buf, rows, sem, bnc, accum):
  c = lax.axis_index("c")
  s = lax.axis_index("s")
  for k in range(NSL // NC):
    sid = c * (NSL // NC) + k
    pltpu.sync_copy(z32, bnc)
    pltpu.sync_copy(bnc, accum.at[pl.ds(s * STRIPE, STRIPE), :])
    plsc.subcore_barrier()
    row0 = s * 784

    def step(i, carry):
      off = row0 + i * CH
      pltpu.sync_copy(src2.at[pl.ds(off, CH), :], src_buf)
      pltpu.sync_copy(dst2.at[pl.ds(off, CH), :], dst_buf)
      for j in range(CH):
        for v in range(8):
          idx_buf[pl.ds(v * 16, 16)] = src_buf[j, pl.ds(v * 16, 16)] * 16 + sid
        pltpu.async_copy(x8.at[idx_buf], rows, sem).wait()
        pltpu.sync_copy(rows, accum.at[dst_buf.at[j]], add=True)
      return carry

    lax.fori_loop(0, 98, step, 0)
    plsc.subcore_barrier()
    pltpu.sync_copy(accum.at[pl.ds(s * STRIPE, STRIPE), :], bnc)
    pltpu.sync_copy(bnc, out.at[sid, pl.ds(s * STRIPE, STRIPE), :])
    plsc.subcore_barrier()


_sc_agg8 = pl.kernel(
    _agg8_body,
    out_type=jax.ShapeDtypeStruct((NSL, NT, SL), jnp.float32),
    mesh=_MESH,
    compiler_params=_SC_PARAMS,
    scratch_types=[
        pltpu.VMEM((CH, 128), jnp.int32),
        pltpu.VMEM((CH, 128), jnp.int32),
        pltpu.VMEM((128,), jnp.int32),
        pltpu.VMEM((128, SL), jnp.float32),
        pltpu.SemaphoreType.DMA,
        pltpu.VMEM((STRIPE, SL), jnp.float32),
        pltpu.VMEM_SHARED((NT, SL), jnp.float32),
    ],
)


def _t0_body(degp_ref, a_ref, b_ref):
  d = degp_ref[...]
  a_ref[...] = lax.rsqrt(jnp.maximum(d[0, 0] + d[1, 0], 1.0))
  b_ref[...] = lax.rsqrt(jnp.maximum(d[0, 1] + d[1, 1], 1.0))


def _t1_body(x_ref, a_ref, o_ref):
  o_ref[...] = x_ref[...] * a_ref[...]


def _t2_body(aggp_ref, b_ref, a_ref, w1_ref, b1_ref, wg1_ref,
             h1_ref, y1_ref):
  p = aggp_ref[0, :, :SL] + aggp_ref[1, :, :SL]
  xagg = p * b_ref[...]
  z = jnp.dot(xagg, w1_ref[...], preferred_element_type=jnp.float32)
  h1 = jnp.maximum(z + b1_ref[...], 0.0)
  h1_ref[...] = h1
  y = jnp.dot(h1, wg1_ref[...], preferred_element_type=jnp.float32)
  y1_ref[...] = y * a_ref[...]


def _t3_body(agg_ref, hp_ref, b_ref, a_ref, bg_ref, wg_ref,
             h_ref, y_ref):
  h = jnp.maximum(agg_ref[...] * b_ref[...] + bg_ref[...], 0.0) + hp_ref[...]
  h_ref[...] = h
  y = jnp.dot(h, wg_ref[...], preferred_element_type=jnp.float32)
  y_ref[...] = y * a_ref[...]


def _t4_body(agg_ref, hp_ref, b_ref, bg_ref, wout_ref, bout_ref,
             wc1_ref, bc1_ref, wc2_ref, bc2_ref, wc3_ref, bc3_ref,
             wc4_ref, bc4_ref, out_ref, acc_ref, *, bs):
  i = pl.program_id(0)
  h3 = jnp.maximum(agg_ref[...] * b_ref[...] + bg_ref[...], 0.0) + hp_ref[...]
  feat = jnp.dot(h3, wout_ref[...], preferred_element_type=jnp.float32)
  feat = jnp.maximum(feat + bout_ref[...], 0.0)
  rid = lax.broadcasted_iota(jnp.int32, (bs, 1), 0) + i * bs
  feat = jnp.where(rid < N, feat, 0.0)
  psum = jnp.sum(feat, axis=0, keepdims=True)

  @pl.when(i == 0)
  def _():
    acc_ref[...] = psum

  @pl.when(i > 0)
  def _():
    acc_ref[...] = acc_ref[...] + psum

  @pl.when(i == NT // bs - 1)
  def _():
    pooled = acc_ref[...] * (1.0 / N)
    z = jnp.maximum(
        jnp.dot(pooled, wc1_ref[...], preferred_element_type=jnp.float32)
        + bc1_ref[...], 0.0)
    z = jnp.maximum(
        jnp.dot(z, wc2_ref[...], preferred_element_type=jnp.float32)
        + bc2_ref[...], 0.0)
    z = jnp.maximum(
        jnp.dot(z, wc3_ref[...], preferred_element_type=jnp.float32)
        + bc3_ref[...], 0.0)
    o = jnp.dot(z, wc4_ref[...], preferred_element_type=jnp.float32)
    out_ref[...] = jnp.broadcast_to(o + bc4_ref[...], (8, 128))


def _full(shape):
  return pl.BlockSpec(shape, lambda i: tuple(0 for _ in shape))


def kernel(nodes, edges, W1, b1, Wg1, bg1, Wg2, bg2, Wout, bout,
           Wc1, bc1, Wc2, bc2, Wc3, bc3, Wc4, bc4):
  f32 = jnp.float32
  src = edges[0].astype(jnp.int32).reshape(NC * NS, E // (NC * NS))
  dst = edges[1].astype(jnp.int32).reshape(NC * NS, E // (NC * NS))
  src2 = jnp.pad(src, ((0, 0), (0, 176)),
                 constant_values=DUMMY).reshape(EP_ROWS, 128)
  dst2 = jnp.pad(dst, ((0, 0), (0, 176)),
                 constant_values=DUMMY).reshape(EP_ROWS, 128)
  zdeg = jnp.zeros((DSTRIPE,), f32)
  z32 = jnp.zeros((STRIPE, SL), f32)

  degp = _sc_degrees(src2, dst2, zdeg)
  a2d, b2d = pl.pallas_call(
      _t0_body,
      out_shape=(jax.ShapeDtypeStruct((391, 128), f32),
                 jax.ShapeDtypeStruct((391, 128), f32)),
  )(degp.reshape(NC, 2, 391, 128))
  a = a2d.reshape(NT, 1)
  b = b2d.reshape(NT, 1)

  nodes128 = jnp.pad(nodes, ((0, NT - N), (0, 128 - nodes.shape[1])))
  bs = 3128
  grid = (NT // bs,)
  xs0 = pl.pallas_call(
      _t1_body,
      grid=grid,
      in_specs=[pl.BlockSpec((bs, 128), lambda i: (i, 0)),
                pl.BlockSpec((bs, 1), lambda i: (i, 0))],
      out_specs=pl.BlockSpec((bs, 128), lambda i: (i, 0)),
      out_shape=jax.ShapeDtypeStruct((NT, 128), f32),
  )(nodes128, a)
  aggp0 = _sc_agg1(xs0.reshape(8 * NT, SL), src2, dst2, z32)

  W1p = jnp.pad(W1, ((0, SL - W1.shape[0]), (0, 0)))
  h1, y1 = pl.pallas_call(
      _t2_body,
      grid=grid,
      in_specs=[
          pl.BlockSpec((NC, bs, 128), lambda i: (0, i, 0)),
          pl.BlockSpec((bs, 1), lambda i: (i, 0)),
          pl.BlockSpec((bs, 1), lambda i: (i, 0)),
          _full((SL, HID)),
          _full((1, HID)),
          _full((HID, HID)),
      ],
      out_specs=(pl.BlockSpec((bs, HID), lambda i: (i, 0)),
                 pl.BlockSpec((bs, HID), lambda i: (i, 0))),
      out_shape=(jax.ShapeDtypeStruct((NT, HID), f32),
                 jax.ShapeDtypeStruct((NT, HID), f32)),
  )(aggp0, b, a, W1p, b1[None, :], Wg1)

  def mid_layer(y, hprev, bg, Wnext):
    aggr = _sc_agg8(y.reshape(NSL * NT, SL), src2, dst2, z32)
    agg_cat = aggr.transpose(1, 0, 2).reshape(NT, HID)
    return pl.pallas_call(
        _t3_body,
        grid=grid,
        in_specs=[
            pl.BlockSpec((bs, HID), lambda i: (i, 0)),
            pl.BlockSpec((bs, HID), lambda i: (i, 0)),
            pl.BlockSpec((bs, 1), lambda i: (i, 0)),
            pl.BlockSpec((bs, 1), lambda i: (i, 0)),
            _full((1, HID)),
            _full((HID, HID)),
        ],
        out_specs=(pl.BlockSpec((bs, HID), lambda i: (i, 0)),
                   pl.BlockSpec((bs, HID), lambda i: (i, 0))),
        out_shape=(jax.ShapeDtypeStruct((NT, HID), f32),
                   jax.ShapeDtypeStruct((NT, HID), f32)),
    )(agg_cat, hprev, b, a, bg[None, :], Wnext)

  h2, y2 = mid_layer(y1, h1, bg1, Wg2)

  aggr3 = _sc_agg8(y2.reshape(NSL * NT, SL), src2, dst2, z32)
  agg3 = aggr3.transpose(1, 0, 2).reshape(NT, HID)

  bs4 = 3128
  Wc4p = jnp.pad(Wc4, ((0, 0), (0, 128 - Wc4.shape[1])))
  bc4p = jnp.pad(bc4, (0, 128 - bc4.shape[0]))
  out8 = pl.pallas_call(
      functools.partial(_t4_body, bs=bs4),
      grid=(NT // bs4,),
      in_specs=[
          pl.BlockSpec((bs4, HID), lambda i: (i, 0)),
          pl.BlockSpec((bs4, HID), lambda i: (i, 0)),
          pl.BlockSpec((bs4, 1), lambda i: (i, 0)),
          _full((1, HID)),
          _full((HID, 512)),
          _full((1, 512)),
          _full((512, 1024)),
          _full((1, 1024)),
          _full((1024, 512)),
          _full((1, 512)),
          _full((512, 256)),
          _full((1, 256)),
          _full((256, 128)),
          _full((1, 128)),
      ],
      out_specs=pl.BlockSpec((8, 128), lambda i: (0, 0)),
      out_shape=jax.ShapeDtypeStruct((8, 128), f32),
      scratch_shapes=[pltpu.VMEM((1, 512), f32)],
  )(agg3, h2, b, bg2[None, :], Wout, bout[None, :],
    Wc1, bc1[None, :], Wc2, bc2[None, :], Wc3, bc3[None, :],
    Wc4p, bc4p[None, :])

  return out8[0, :5]


# speedup vs baseline: 8.2103x; 1.9860x over previous
import functools

import jax
import jax.numpy as jnp
from jax import lax
from jax.experimental import pallas as pl
from jax.experimental.pallas import tpu as pltpu
from jax.experimental.pallas import tpu_sc as plsc

N = 50000
E = 1600000
NT = 50048
DUMMY = 50040
EP_ROWS = 12544
HID = 256
SL = 16
NSL = HID // SL
NC = 2
NS = 16
CH = 8
STRIPE = NT // NS
DSTRIPE = NT // 8

_MESH = plsc.VectorSubcoreMesh(
    core_axis_name="c", subcore_axis_name="s", num_cores=NC, num_subcores=NS)
_SC_PARAMS = pltpu.CompilerParams(use_tc_tiling_on_sc=False)


def _deg_body(src2, dst2, zdeg, out, src_buf, dst_buf, ones, bnc, dsem,
              h_out, h_in):
  c = lax.axis_index("c")
  s = lax.axis_index("s")
  w = c * NS + s
  for i in range(8):
    ones[pl.ds(i * 16, 16)] = jnp.full((16,), 1.0, jnp.float32)
  pltpu.sync_copy(zdeg, bnc)

  @pl.when(s < 8)
  def _():
    pltpu.sync_copy(bnc, h_out.at[pl.ds(s * DSTRIPE, DSTRIPE)])

  @pl.when(s >= 8)
  def _():
    pltpu.sync_copy(bnc, h_in.at[pl.ds((s - 8) * DSTRIPE, DSTRIPE)])

  plsc.subcore_barrier()
  row0 = w * 392

  def step(i, carry):
    off = row0 + i * CH
    pltpu.sync_copy(src2.at[pl.ds(off, CH), :], src_buf)
    pltpu.sync_copy(dst2.at[pl.ds(off, CH), :], dst_buf)
    scatters = []
    for j in range(CH):
      scatters.append(
          pltpu.async_copy(ones, h_out.at[src_buf.at[j]], dsem, add=True))
      scatters.append(
          pltpu.async_copy(ones, h_in.at[dst_buf.at[j]], dsem, add=True))
    for sc in scatters:
      sc.wait()
    return carry

  lax.fori_loop(0, 49, step, 0)
  plsc.subcore_barrier()
  for half, hist in ((0, h_out), (1, h_in)):
    base = c * 2 * NT + half * NT
    pltpu.sync_copy(hist.at[pl.ds(s * STRIPE, STRIPE)],
                    bnc.at[pl.ds(0, STRIPE)])
    pltpu.sync_copy(bnc.at[pl.ds(0, STRIPE)],
                    out.at[pl.ds(base + s * STRIPE, STRIPE)])


_sc_degrees = pl.kernel(
    _deg_body,
    out_type=jax.ShapeDtypeStruct((NC * 2 * NT,), jnp.float32),
    mesh=_MESH,
    compiler_params=_SC_PARAMS,
    scratch_types=[
        pltpu.VMEM((CH, 128), jnp.int32),
        pltpu.VMEM((CH, 128), jnp.int32),
        pltpu.VMEM((128,), jnp.float32),
        pltpu.VMEM((DSTRIPE,), jnp.float32),
        pltpu.SemaphoreType.DMA,
        pltpu.VMEM_SHARED((NT,), jnp.float32),
        pltpu.VMEM_SHARED((NT,), jnp.float32),
    ],
)


def _agg1_body(x4, src2, dst2, z32, out,
               src_buf, dst_buf, idx_buf, rows, sem, sem2, bnc, accum):
  c = lax.axis_index("c")
  s = lax.axis_index("s")
  w = c * NS + s
  pltpu.sync_copy(z32, bnc)
  pltpu.sync_copy(bnc, accum.at[pl.ds(s * STRIPE, STRIPE), :])
  plsc.subcore_barrier()
  row0 = w * 392

  def step(i, carry):
    off = row0 + i * CH
    pltpu.sync_copy(src2.at[pl.ds(off, CH), :], src_buf)
    pltpu.sync_copy(dst2.at[pl.ds(off, CH), :], dst_buf)
    for j in range(CH):
      for v in range(8):
        idx_buf[j, pl.ds(v * 16, 16)] = src_buf[j, pl.ds(v * 16, 16)] * 8
    gathers = [
        pltpu.async_copy(x4.at[idx_buf.at[j]], rows.at[j], sem)
        for j in range(CH)]
    for g in gathers:
      g.wait()
    scatters = [
        pltpu.async_copy(rows.at[j], accum.at[dst_buf.at[j]], sem2, add=True)
        for j in range(CH)]
    for sc in scatters:
      sc.wait()
    return carry

  lax.fori_loop(0, 49, step, 0)
  plsc.subcore_barrier()
  pltpu.sync_copy(accum.at[pl.ds(s * STRIPE, STRIPE), :], bnc)
  pltpu.sync_copy(bnc, out.at[c, pl.ds(s * STRIPE, STRIPE), pl.ds(0, SL)])


_sc_agg1 = pl.kernel(
    _agg1_body,
    out_type=jax.ShapeDtypeStruct((NC, NT, 128), jnp.float32),
    mesh=_MESH,
    compiler_params=_SC_PARAMS,
    scratch_types=[
        pltpu.VMEM((CH, 128), jnp.int32),
        pltpu.VMEM((CH, 128), jnp.int32),
        pltpu.VMEM((CH, 128), jnp.int32),
        pltpu.VMEM((CH, 128, SL), jnp.float32),
        pltpu.SemaphoreType.DMA,
        pltpu.SemaphoreType.DMA,
        pltpu.VMEM((STRIPE, SL), jnp.float32),
        pltpu.VMEM_SHARED((NT, SL), jnp.float32),
    ],
)


def _agg8_body(x8, src2, dst2, z32, out,
               src_buf, dst_buf, idx_buf, rows, sem, sem2, bnc, accum):
  c = lax.axis_index("c")
  s = lax.axis_index("s")
  for k in range(NSL // NC):
    sid = c * (NSL // NC) + k
    pltpu.sync_copy(z32, bnc)
    pltpu.sync_copy(bnc, accum.at[pl.ds(s * STRIPE, STRIPE), :])
    plsc.subcore_barrier()
    row0 = s * 784

    def step(i, carry):
      off = row0 + i * CH
      pltpu.sync_copy(src2.at[pl.ds(off, CH), :], src_buf)
      pltpu.sync_copy(dst2.at[pl.ds(off, CH), :], dst_buf)
      for j in range(CH):
        for v in range(8):
          idx_buf[j, pl.ds(v * 16, 16)] = (
              src_buf[j, pl.ds(v * 16, 16)] * 16 + sid)
      gathers = [
          pltpu.async_copy(x8.at[idx_buf.at[j]], rows.at[j], sem)
          for j in range(CH)]
      for g in gathers:
        g.wait()
      scatters = [
          pltpu.async_copy(rows.at[j], accum.at[dst_buf.at[j]], sem2,
                           add=True)
          for j in range(CH)]
      for sc in scatters:
        sc.wait()
      return carry

    lax.fori_loop(0, 98, step, 0)
    plsc.subcore_barrier()
    pltpu.sync_copy(accum.at[pl.ds(s * STRIPE, STRIPE), :], bnc)
    pltpu.sync_copy(bnc, out.at[sid, pl.ds(s * STRIPE, STRIPE), :])
    plsc.subcore_barrier()


_sc_agg8 = pl.kernel(
    _agg8_body,
    out_type=jax.ShapeDtypeStruct((NSL, NT, SL), jnp.float32),
    mesh=_MESH,
    compiler_params=_SC_PARAMS,
    scratch_types=[
        pltpu.VMEM((CH, 128), jnp.int32),
        pltpu.VMEM((CH, 128), jnp.int32),
        pltpu.VMEM((CH, 128), jnp.int32),
        pltpu.VMEM((CH, 128, SL), jnp.float32),
        pltpu.SemaphoreType.DMA,
        pltpu.SemaphoreType.DMA,
        pltpu.VMEM((STRIPE, SL), jnp.float32),
        pltpu.VMEM_SHARED((NT, SL), jnp.float32),
    ],
)


def _t0_body(degp_ref, a_ref, b_ref):
  d = degp_ref[...]
  a_ref[...] = lax.rsqrt(jnp.maximum(d[0, 0] + d[1, 0], 1.0))
  b_ref[...] = lax.rsqrt(jnp.maximum(d[0, 1] + d[1, 1], 1.0))


def _t1_body(x_ref, a_ref, o_ref):
  o_ref[...] = x_ref[...] * a_ref[...]


def _t2_body(aggp_ref, b_ref, a_ref, w1_ref, b1_ref, wg1_ref,
             h1_ref, y1_ref):
  p = aggp_ref[0, :, :SL] + aggp_ref[1, :, :SL]
  xagg = p * b_ref[...]
  z = jnp.dot(xagg, w1_ref[...], preferred_element_type=jnp.float32)
  h1 = jnp.maximum(z + b1_ref[...], 0.0)
  h1_ref[...] = h1
  y = jnp.dot(h1, wg1_ref[...], preferred_element_type=jnp.float32)
  y1_ref[...] = y * a_ref[...]


def _t3_body(agg_ref, hp_ref, b_ref, a_ref, bg_ref, wg_ref,
             h_ref, y_ref):
  h = jnp.maximum(agg_ref[...] * b_ref[...] + bg_ref[...], 0.0) + hp_ref[...]
  h_ref[...] = h
  y = jnp.dot(h, wg_ref[...], preferred_element_type=jnp.float32)
  y_ref[...] = y * a_ref[...]


def _t4_body(agg_ref, hp_ref, b_ref, bg_ref, wout_ref, bout_ref,
             wc1_ref, bc1_ref, wc2_ref, bc2_ref, wc3_ref, bc3_ref,
             wc4_ref, bc4_ref, out_ref, acc_ref, *, bs):
  i = pl.program_id(0)
  h3 = jnp.maximum(agg_ref[...] * b_ref[...] + bg_ref[...], 0.0) + hp_ref[...]
  feat = jnp.dot(h3, wout_ref[...], preferred_element_type=jnp.float32)
  feat = jnp.maximum(feat + bout_ref[...], 0.0)
  rid = lax.broadcasted_iota(jnp.int32, (bs, 1), 0) + i * bs
  feat = jnp.where(rid < N, feat, 0.0)
  psum = jnp.sum(feat, axis=0, keepdims=True)

  @pl.when(i == 0)
  def _():
    acc_ref[...] = psum

  @pl.when(i > 0)
  def _():
    acc_ref[...] = acc_ref[...] + psum

  @pl.when(i == NT // bs - 1)
  def _():
    pooled = acc_ref[...] * (1.0 / N)
    z = jnp.maximum(
        jnp.dot(pooled, wc1_ref[...], preferred_element_type=jnp.float32)
        + bc1_ref[...], 0.0)
    z = jnp.maximum(
        jnp.dot(z, wc2_ref[...], preferred_element_type=jnp.float32)
        + bc2_ref[...], 0.0)
    z = jnp.maximum(
        jnp.dot(z, wc3_ref[...], preferred_element_type=jnp.float32)
        + bc3_ref[...], 0.0)
    o = jnp.dot(z, wc4_ref[...], preferred_element_type=jnp.float32)
    out_ref[...] = jnp.broadcast_to(o + bc4_ref[...], (8, 128))


def _full(shape):
  return pl.BlockSpec(shape, lambda i: tuple(0 for _ in shape))


def kernel(nodes, edges, W1, b1, Wg1, bg1, Wg2, bg2, Wout, bout,
           Wc1, bc1, Wc2, bc2, Wc3, bc3, Wc4, bc4):
  f32 = jnp.float32
  src = edges[0].astype(jnp.int32).reshape(NC * NS, E // (NC * NS))
  dst = edges[1].astype(jnp.int32).reshape(NC * NS, E // (NC * NS))
  src2 = jnp.pad(src, ((0, 0), (0, 176)),
                 constant_values=DUMMY).reshape(EP_ROWS, 128)
  dst2 = jnp.pad(dst, ((0, 0), (0, 176)),
                 constant_values=DUMMY).reshape(EP_ROWS, 128)
  zdeg = jnp.zeros((DSTRIPE,), f32)
  z32 = jnp.zeros((STRIPE, SL), f32)

  degp = _sc_degrees(src2, dst2, zdeg)
  a2d, b2d = pl.pallas_call(
      _t0_body,
      out_shape=(jax.ShapeDtypeStruct((391, 128), f32),
                 jax.ShapeDtypeStruct((391, 128), f32)),
  )(degp.reshape(NC, 2, 391, 128))
  a = a2d.reshape(NT, 1)
  b = b2d.reshape(NT, 1)

  nodes128 = jnp.pad(nodes, ((0, NT - N), (0, 128 - nodes.shape[1])))
  bs = 3128
  grid = (NT // bs,)
  xs0 = pl.pallas_call(
      _t1_body,
      grid=grid,
      in_specs=[pl.BlockSpec((bs, 128), lambda i: (i, 0)),
                pl.BlockSpec((bs, 1), lambda i: (i, 0))],
      out_specs=pl.BlockSpec((bs, 128), lambda i: (i, 0)),
      out_shape=jax.ShapeDtypeStruct((NT, 128), f32),
  )(nodes128, a)
  aggp0 = _sc_agg1(xs0.reshape(8 * NT, SL), src2, dst2, z32)

  W1p = jnp.pad(W1, ((0, SL - W1.shape[0]), (0, 0)))
  h1, y1 = pl.pallas_call(
      _t2_body,
      grid=grid,
      in_specs=[
          pl.BlockSpec((NC, bs, 128), lambda i: (0, i, 0)),
          pl.BlockSpec((bs, 1), lambda i: (i, 0)),
          pl.BlockSpec((bs, 1), lambda i: (i, 0)),
          _full((SL, HID)),
          _full((1, HID)),
          _full((HID, HID)),
      ],
      out_specs=(pl.BlockSpec((bs, HID), lambda i: (i, 0)),
                 pl.BlockSpec((bs, HID), lambda i: (i, 0))),
      out_shape=(jax.ShapeDtypeStruct((NT, HID), f32),
                 jax.ShapeDtypeStruct((NT, HID), f32)),
  )(aggp0, b, a, W1p, b1[None, :], Wg1)

  def mid_layer(y, hprev, bg, Wnext):
    aggr = _sc_agg8(y.reshape(NSL * NT, SL), src2, dst2, z32)
    agg_cat = aggr.transpose(1, 0, 2).reshape(NT, HID)
    return pl.pallas_call(
        _t3_body,
        grid=grid,
        in_specs=[
            pl.BlockSpec((bs, HID), lambda i: (i, 0)),
            pl.BlockSpec((bs, HID), lambda i: (i, 0)),
            pl.BlockSpec((bs, 1), lambda i: (i, 0)),
            pl.BlockSpec((bs, 1), lambda i: (i, 0)),
            _full((1, HID)),
            _full((HID, HID)),
        ],
        out_specs=(pl.BlockSpec((bs, HID), lambda i: (i, 0)),
                   pl.BlockSpec((bs, HID), lambda i: (i, 0))),
        out_shape=(jax.ShapeDtypeStruct((NT, HID), f32),
                   jax.ShapeDtypeStruct((NT, HID), f32)),
    )(agg_cat, hprev, b, a, bg[None, :], Wnext)

  h2, y2 = mid_layer(y1, h1, bg1, Wg2)

  aggr3 = _sc_agg8(y2.reshape(NSL * NT, SL), src2, dst2, z32)
  agg3 = aggr3.transpose(1, 0, 2).reshape(NT, HID)

  bs4 = 3128
  Wc4p = jnp.pad(Wc4, ((0, 0), (0, 128 - Wc4.shape[1])))
  bc4p = jnp.pad(bc4, (0, 128 - bc4.shape[0]))
  out8 = pl.pallas_call(
      functools.partial(_t4_body, bs=bs4),
      grid=(NT // bs4,),
      in_specs=[
          pl.BlockSpec((bs4, HID), lambda i: (i, 0)),
          pl.BlockSpec((bs4, HID), lambda i: (i, 0)),
          pl.BlockSpec((bs4, 1), lambda i: (i, 0)),
          _full((1, HID)),
          _full((HID, 512)),
          _full((1, 512)),
          _full((512, 1024)),
          _full((1, 1024)),
          _full((1024, 512)),
          _full((1, 512)),
          _full((512, 256)),
          _full((1, 256)),
          _full((256, 128)),
          _full((1, 128)),
      ],
      out_specs=pl.BlockSpec((8, 128), lambda i: (0, 0)),
      out_shape=jax.ShapeDtypeStruct((8, 128), f32),
      scratch_shapes=[pltpu.VMEM((1, 512), f32)],
  )(agg3, h2, b, bg2[None, :], Wout, bout[None, :],
    Wc1, bc1[None, :], Wc2, bc2[None, :], Wc3, bc3[None, :],
    Wc4p, bc4p[None, :])

  return out8[0, :5]


# speedup vs baseline: 11.1163x; 1.3539x over previous
import functools

import jax
import jax.numpy as jnp
from jax import lax
from jax.experimental import pallas as pl
from jax.experimental.pallas import tpu as pltpu
from jax.experimental.pallas import tpu_sc as plsc

N = 50000
E = 1600000
NT = 50048
DUMMY = 50040
EP_ROWS = 12544
HID = 256
SL = 16
NSL = HID // SL
NC = 2
NS = 16
CH = 8
STRIPE = NT // NS
DSTRIPE = NT // 8

_MESH = plsc.VectorSubcoreMesh(
    core_axis_name="c", subcore_axis_name="s", num_cores=NC, num_subcores=NS)
_SC_PARAMS = pltpu.CompilerParams(use_tc_tiling_on_sc=False)


def _deg_body(src2, dst2, zdeg, out, src_buf, dst_buf, ones, bnc, dsem,
              h_out, h_in):
  c = lax.axis_index("c")
  s = lax.axis_index("s")
  w = c * NS + s
  for i in range(8):
    ones[pl.ds(i * 16, 16)] = jnp.full((16,), 1.0, jnp.float32)
  pltpu.sync_copy(zdeg, bnc)

  @pl.when(s < 8)
  def _():
    pltpu.sync_copy(bnc, h_out.at[pl.ds(s * DSTRIPE, DSTRIPE)])

  @pl.when(s >= 8)
  def _():
    pltpu.sync_copy(bnc, h_in.at[pl.ds((s - 8) * DSTRIPE, DSTRIPE)])

  plsc.subcore_barrier()
  row0 = w * 392

  def step(i, carry):
    off = row0 + i * CH
    pltpu.sync_copy(src2.at[pl.ds(off, CH), :], src_buf)
    pltpu.sync_copy(dst2.at[pl.ds(off, CH), :], dst_buf)
    scatters = []
    for j in range(CH):
      scatters.append(
          pltpu.async_copy(ones, h_out.at[src_buf.at[j]], dsem, add=True))
      scatters.append(
          pltpu.async_copy(ones, h_in.at[dst_buf.at[j]], dsem, add=True))
    for sc in scatters:
      sc.wait()
    return carry

  lax.fori_loop(0, 49, step, 0)
  plsc.subcore_barrier()
  for half, hist in ((0, h_out), (1, h_in)):
    base = c * 2 * NT + half * NT
    pltpu.sync_copy(hist.at[pl.ds(s * STRIPE, STRIPE)],
                    bnc.at[pl.ds(0, STRIPE)])
    pltpu.sync_copy(bnc.at[pl.ds(0, STRIPE)],
                    out.at[pl.ds(base + s * STRIPE, STRIPE)])


_sc_degrees = pl.kernel(
    _deg_body,
    out_type=jax.ShapeDtypeStruct((NC * 2 * NT,), jnp.float32),
    mesh=_MESH,
    compiler_params=_SC_PARAMS,
    scratch_types=[
        pltpu.VMEM((CH, 128), jnp.int32),
        pltpu.VMEM((CH, 128), jnp.int32),
        pltpu.VMEM((128,), jnp.float32),
        pltpu.VMEM((DSTRIPE,), jnp.float32),
        pltpu.SemaphoreType.DMA,
        pltpu.VMEM_SHARED((NT,), jnp.float32),
        pltpu.VMEM_SHARED((NT,), jnp.float32),
    ],
)


def _agg1_body(x4, src2, dst2, z32, out,
               src_buf, dst_buf, idx_buf, rows, sem, sem2, sem_t, bnc,
               accum):
  c = lax.axis_index("c")
  s = lax.axis_index("s")
  w = c * NS + s
  nblk = 49
  pltpu.sync_copy(z32, bnc)
  pltpu.sync_copy(bnc, accum.at[pl.ds(s * STRIPE, STRIPE), :])
  plsc.subcore_barrier()
  row0 = w * 392

  pltpu.async_copy(src2.at[pl.ds(row0, CH), :], src_buf.at[0], sem_t)
  pltpu.async_copy(dst2.at[pl.ds(row0, CH), :], dst_buf.at[0], sem_t)

  def step(i, carry):
    p = lax.rem(i, 2)
    q = 1 - p
    pltpu.make_async_copy(src2.at[pl.ds(0, CH), :], src_buf.at[p],
                          sem_t).wait()
    pltpu.make_async_copy(dst2.at[pl.ds(0, CH), :], dst_buf.at[p],
                          sem_t).wait()
    off_n = row0 + jnp.minimum(i + 1, nblk - 1) * CH
    pltpu.async_copy(src2.at[pl.ds(off_n, CH), :], src_buf.at[q], sem_t)
    pltpu.async_copy(dst2.at[pl.ds(off_n, CH), :], dst_buf.at[q], sem_t)
    for j in range(CH):
      for v in range(8):
        idx_buf[j, pl.ds(v * 16, 16)] = src_buf[p, j, pl.ds(v * 16, 16)] * 8
    gathers = [
        pltpu.async_copy(x4.at[idx_buf.at[j]], rows.at[j], sem)
        for j in range(CH)]
    scatters = []
    for j in range(CH):
      gathers[j].wait()
      scatters.append(
          pltpu.async_copy(rows.at[j], accum.at[dst_buf.at[p, j]],
                           sem2, add=True))
    for sc in scatters:
      sc.wait()
    return carry

  lax.fori_loop(0, nblk, step, 0)
  pltpu.make_async_copy(src2.at[pl.ds(0, CH), :], src_buf.at[0],
                        sem_t).wait()
  pltpu.make_async_copy(dst2.at[pl.ds(0, CH), :], dst_buf.at[0],
                        sem_t).wait()
  plsc.subcore_barrier()
  pltpu.sync_copy(accum.at[pl.ds(s * STRIPE, STRIPE), :], bnc)
  pltpu.sync_copy(bnc, out.at[c, pl.ds(s * STRIPE, STRIPE), pl.ds(0, SL)])


_sc_agg1 = pl.kernel(
    _agg1_body,
    out_type=jax.ShapeDtypeStruct((NC, NT, 128), jnp.float32),
    mesh=_MESH,
    compiler_params=_SC_PARAMS,
    scratch_types=[
        pltpu.VMEM((2, CH, 128), jnp.int32),
        pltpu.VMEM((2, CH, 128), jnp.int32),
        pltpu.VMEM((CH, 128), jnp.int32),
        pltpu.VMEM((CH, 128, SL), jnp.float32),
        pltpu.SemaphoreType.DMA,
        pltpu.SemaphoreType.DMA,
        pltpu.SemaphoreType.DMA,
        pltpu.VMEM((STRIPE, SL), jnp.float32),
        pltpu.VMEM_SHARED((NT, SL), jnp.float32),
    ],
)


def _agg8_body(x8, src2, dst2, z32, out,
               src_buf, dst_buf, idx_buf, rows, sem, sem2, sem_t, bnc,
               accum):
  c = lax.axis_index("c")
  s = lax.axis_index("s")
  nblk = 98
  for k in range(NSL // NC):
    sid = c * (NSL // NC) + k
    pltpu.sync_copy(z32, bnc)
    pltpu.sync_copy(bnc, accum.at[pl.ds(s * STRIPE, STRIPE), :])
    plsc.subcore_barrier()
    row0 = s * 784

    pltpu.async_copy(src2.at[pl.ds(row0, CH), :], src_buf.at[0], sem_t)
    pltpu.async_copy(dst2.at[pl.ds(row0, CH), :], dst_buf.at[0], sem_t)

    def step(i, carry):
      p = lax.rem(i, 2)
      q = 1 - p
      pltpu.make_async_copy(src2.at[pl.ds(0, CH), :], src_buf.at[p],
                            sem_t).wait()
      pltpu.make_async_copy(dst2.at[pl.ds(0, CH), :], dst_buf.at[p],
                            sem_t).wait()
      off_n = row0 + jnp.minimum(i + 1, nblk - 1) * CH
      pltpu.async_copy(src2.at[pl.ds(off_n, CH), :], src_buf.at[q], sem_t)
      pltpu.async_copy(dst2.at[pl.ds(off_n, CH), :], dst_buf.at[q], sem_t)
      for j in range(CH):
        for v in range(8):
          idx_buf[j, pl.ds(v * 16, 16)] = (
              src_buf[p, j, pl.ds(v * 16, 16)] * 16 + sid)
      gathers = [
          pltpu.async_copy(x8.at[idx_buf.at[j]], rows.at[j], sem)
          for j in range(CH)]
      scatters = []
      for j in range(CH):
        gathers[j].wait()
        scatters.append(
            pltpu.async_copy(rows.at[j], accum.at[dst_buf.at[p, j]],
                             sem2, add=True))
      for sc in scatters:
        sc.wait()
      return carry

    lax.fori_loop(0, nblk, step, 0)
    pltpu.make_async_copy(src2.at[pl.ds(0, CH), :], src_buf.at[0],
                          sem_t).wait()
    pltpu.make_async_copy(dst2.at[pl.ds(0, CH), :], dst_buf.at[0],
                          sem_t).wait()
    plsc.subcore_barrier()
    pltpu.sync_copy(accum.at[pl.ds(s * STRIPE, STRIPE), :], bnc)
    pltpu.sync_copy(bnc, out.at[sid, pl.ds(s * STRIPE, STRIPE), :])
    plsc.subcore_barrier()


_sc_agg8 = pl.kernel(
    _agg8_body,
    out_type=jax.ShapeDtypeStruct((NSL, NT, SL), jnp.float32),
    mesh=_MESH,
    compiler_params=_SC_PARAMS,
    scratch_types=[
        pltpu.VMEM((2, CH, 128), jnp.int32),
        pltpu.VMEM((2, CH, 128), jnp.int32),
        pltpu.VMEM((CH, 128), jnp.int32),
        pltpu.VMEM((CH, 128, SL), jnp.float32),
        pltpu.SemaphoreType.DMA,
        pltpu.SemaphoreType.DMA,
        pltpu.SemaphoreType.DMA,
        pltpu.VMEM((STRIPE, SL), jnp.float32),
        pltpu.VMEM_SHARED((NT, SL), jnp.float32),
    ],
)


def _t0_body(degp_ref, a_ref, b_ref):
  d = degp_ref[...]
  a_ref[...] = lax.rsqrt(jnp.maximum(d[0, 0] + d[1, 0], 1.0))
  b_ref[...] = lax.rsqrt(jnp.maximum(d[0, 1] + d[1, 1], 1.0))


def _t1_body(x_ref, a_ref, o_ref):
  o_ref[...] = x_ref[...] * a_ref[...]


def _t2_body(aggp_ref, b_ref, a_ref, w1_ref, b1_ref, wg1_ref,
             h1_ref, y1_ref):
  p = aggp_ref[0, :, :SL] + aggp_ref[1, :, :SL]
  xagg = p * b_ref[...]
  z = jnp.dot(xagg, w1_ref[...], preferred_element_type=jnp.float32)
  h1 = jnp.maximum(z + b1_ref[...], 0.0)
  h1_ref[...] = h1
  y = jnp.dot(h1, wg1_ref[...], preferred_element_type=jnp.float32)
  y1_ref[...] = y * a_ref[...]


def _t3_body(agg_ref, hp_ref, b_ref, a_ref, bg_ref, wg_ref,
             h_ref, y_ref):
  h = jnp.maximum(agg_ref[...] * b_ref[...] + bg_ref[...], 0.0) + hp_ref[...]
  h_ref[...] = h
  y = jnp.dot(h, wg_ref[...], preferred_element_type=jnp.float32)
  y_ref[...] = y * a_ref[...]


def _t4_body(agg_ref, hp_ref, b_ref, bg_ref, wout_ref, bout_ref,
             wc1_ref, bc1_ref, wc2_ref, bc2_ref, wc3_ref, bc3_ref,
             wc4_ref, bc4_ref, out_ref, acc_ref, *, bs):
  i = pl.program_id(0)
  h3 = jnp.maximum(agg_ref[...] * b_ref[...] + bg_ref[...], 0.0) + hp_ref[...]
  feat = jnp.dot(h3, wout_ref[...], preferred_element_type=jnp.float32)
  feat = jnp.maximum(feat + bout_ref[...], 0.0)
  rid = lax.broadcasted_iota(jnp.int32, (bs, 1), 0) + i * bs
  feat = jnp.where(rid < N, feat, 0.0)
  psum = jnp.sum(feat, axis=0, keepdims=True)

  @pl.when(i == 0)
  def _():
    acc_ref[...] = psum

  @pl.when(i > 0)
  def _():
    acc_ref[...] = acc_ref[...] + psum

  @pl.when(i == NT // bs - 1)
  def _():
    pooled = acc_ref[...] * (1.0 / N)
    z = jnp.maximum(
        jnp.dot(pooled, wc1_ref[...], preferred_element_type=jnp.float32)
        + bc1_ref[...], 0.0)
    z = jnp.maximum(
        jnp.dot(z, wc2_ref[...], preferred_element_type=jnp.float32)
        + bc2_ref[...], 0.0)
    z = jnp.maximum(
        jnp.dot(z, wc3_ref[...], preferred_element_type=jnp.float32)
        + bc3_ref[...], 0.0)
    o = jnp.dot(z, wc4_ref[...], preferred_element_type=jnp.float32)
    out_ref[...] = jnp.broadcast_to(o + bc4_ref[...], (8, 128))


def _full(shape):
  return pl.BlockSpec(shape, lambda i: tuple(0 for _ in shape))


def kernel(nodes, edges, W1, b1, Wg1, bg1, Wg2, bg2, Wout, bout,
           Wc1, bc1, Wc2, bc2, Wc3, bc3, Wc4, bc4):
  f32 = jnp.float32
  src = edges[0].astype(jnp.int32).reshape(NC * NS, E // (NC * NS))
  dst = edges[1].astype(jnp.int32).reshape(NC * NS, E // (NC * NS))
  src2 = jnp.pad(src, ((0, 0), (0, 176)),
                 constant_values=DUMMY).reshape(EP_ROWS, 128)
  dst2 = jnp.pad(dst, ((0, 0), (0, 176)),
                 constant_values=DUMMY).reshape(EP_ROWS, 128)
  zdeg = jnp.zeros((DSTRIPE,), f32)
  z32 = jnp.zeros((STRIPE, SL), f32)

  degp = _sc_degrees(src2, dst2, zdeg)
  a2d, b2d = pl.pallas_call(
      _t0_body,
      out_shape=(jax.ShapeDtypeStruct((391, 128), f32),
                 jax.ShapeDtypeStruct((391, 128), f32)),
  )(degp.reshape(NC, 2, 391, 128))
  a = a2d.reshape(NT, 1)
  b = b2d.reshape(NT, 1)

  nodes128 = jnp.pad(nodes, ((0, NT - N), (0, 128 - nodes.shape[1])))
  bs = 3128
  grid = (NT // bs,)
  xs0 = pl.pallas_call(
      _t1_body,
      grid=grid,
      in_specs=[pl.BlockSpec((bs, 128), lambda i: (i, 0)),
                pl.BlockSpec((bs, 1), lambda i: (i, 0))],
      out_specs=pl.BlockSpec((bs, 128), lambda i: (i, 0)),
      out_shape=jax.ShapeDtypeStruct((NT, 128), f32),
  )(nodes128, a)
  aggp0 = _sc_agg1(xs0.reshape(8 * NT, SL), src2, dst2, z32)

  W1p = jnp.pad(W1, ((0, SL - W1.shape[0]), (0, 0)))
  h1, y1 = pl.pallas_call(
      _t2_body,
      grid=grid,
      in_specs=[
          pl.BlockSpec((NC, bs, 128), lambda i: (0, i, 0)),
          pl.BlockSpec((bs, 1), lambda i: (i, 0)),
          pl.BlockSpec((bs, 1), lambda i: (i, 0)),
          _full((SL, HID)),
          _full((1, HID)),
          _full((HID, HID)),
      ],
      out_specs=(pl.BlockSpec((bs, HID), lambda i: (i, 0)),
                 pl.BlockSpec((bs, HID), lambda i: (i, 0))),
      out_shape=(jax.ShapeDtypeStruct((NT, HID), f32),
                 jax.ShapeDtypeStruct((NT, HID), f32)),
  )(aggp0, b, a, W1p, b1[None, :], Wg1)

  def mid_layer(y, hprev, bg, Wnext):
    aggr = _sc_agg8(y.reshape(NSL * NT, SL), src2, dst2, z32)
    agg_cat = aggr.transpose(1, 0, 2).reshape(NT, HID)
    return pl.pallas_call(
        _t3_body,
        grid=grid,
        in_specs=[
            pl.BlockSpec((bs, HID), lambda i: (i, 0)),
            pl.BlockSpec((bs, HID), lambda i: (i, 0)),
            pl.BlockSpec((bs, 1), lambda i: (i, 0)),
            pl.BlockSpec((bs, 1), lambda i: (i, 0)),
            _full((1, HID)),
            _full((HID, HID)),
        ],
        out_specs=(pl.BlockSpec((bs, HID), lambda i: (i, 0)),
                   pl.BlockSpec((bs, HID), lambda i: (i, 0))),
        out_shape=(jax.ShapeDtypeStruct((NT, HID), f32),
                   jax.ShapeDtypeStruct((NT, HID), f32)),
    )(agg_cat, hprev, b, a, bg[None, :], Wnext)

  h2, y2 = mid_layer(y1, h1, bg1, Wg2)

  aggr3 = _sc_agg8(y2.reshape(NSL * NT, SL), src2, dst2, z32)
  agg3 = aggr3.transpose(1, 0, 2).reshape(NT, HID)

  bs4 = 3128
  Wc4p = jnp.pad(Wc4, ((0, 0), (0, 128 - Wc4.shape[1])))
  bc4p = jnp.pad(bc4, (0, 128 - bc4.shape[0]))
  out8 = pl.pallas_call(
      functools.partial(_t4_body, bs=bs4),
      grid=(NT // bs4,),
      in_specs=[
          pl.BlockSpec((bs4, HID), lambda i: (i, 0)),
          pl.BlockSpec((bs4, HID), lambda i: (i, 0)),
          pl.BlockSpec((bs4, 1), lambda i: (i, 0)),
          _full((1, HID)),
          _full((HID, 512)),
          _full((1, 512)),
          _full((512, 1024)),
          _full((1, 1024)),
          _full((1024, 512)),
          _full((1, 512)),
          _full((512, 256)),
          _full((1, 256)),
          _full((256, 128)),
          _full((1, 128)),
      ],
      out_specs=pl.BlockSpec((8, 128), lambda i: (0, 0)),
      out_shape=jax.ShapeDtypeStruct((8, 128), f32),
      scratch_shapes=[pltpu.VMEM((1, 512), f32)],
  )(agg3, h2, b, bg2[None, :], Wout, bout[None, :],
    Wc1, bc1[None, :], Wc2, bc2[None, :], Wc3, bc3[None, :],
    Wc4p, bc4p[None, :])

  return out8[0, :5]


# speedup vs baseline: 11.5394x; 1.0381x over previous
import functools

import jax
import jax.numpy as jnp
from jax import lax
from jax.experimental import pallas as pl
from jax.experimental.pallas import tpu as pltpu
from jax.experimental.pallas import tpu_sc as plsc

N = 50000
E = 1600000
NT = 50048
DUMMY = 50040
EP_ROWS = 12544
HID = 256
SL = 16
NSL = HID // SL
NC = 2
NS = 16
CH = 8
STRIPE = NT // NS
DSTRIPE = NT // 8

_MESH = plsc.VectorSubcoreMesh(
    core_axis_name="c", subcore_axis_name="s", num_cores=NC, num_subcores=NS)
_SC_PARAMS = pltpu.CompilerParams(use_tc_tiling_on_sc=False)


def _deg_body(src2, dst2, zdeg, out, src_buf, dst_buf, ones, bnc, dsem,
              h_out, h_in):
  c = lax.axis_index("c")
  s = lax.axis_index("s")
  w = c * NS + s
  for i in range(8):
    ones[pl.ds(i * 16, 16)] = jnp.full((16,), 1.0, jnp.float32)
  pltpu.sync_copy(zdeg, bnc)

  @pl.when(s < 8)
  def _():
    pltpu.sync_copy(bnc, h_out.at[pl.ds(s * DSTRIPE, DSTRIPE)])

  @pl.when(s >= 8)
  def _():
    pltpu.sync_copy(bnc, h_in.at[pl.ds((s - 8) * DSTRIPE, DSTRIPE)])

  plsc.subcore_barrier()
  row0 = w * 392

  def step(i, carry):
    off = row0 + i * CH
    pltpu.sync_copy(src2.at[pl.ds(off, CH), :], src_buf)
    pltpu.sync_copy(dst2.at[pl.ds(off, CH), :], dst_buf)
    scatters = []
    for j in range(CH):
      scatters.append(
          pltpu.async_copy(ones, h_out.at[src_buf.at[j]], dsem, add=True))
      scatters.append(
          pltpu.async_copy(ones, h_in.at[dst_buf.at[j]], dsem, add=True))
    for sc in scatters:
      sc.wait()
    return carry

  lax.fori_loop(0, 49, step, 0)
  plsc.subcore_barrier()
  for half, hist in ((0, h_out), (1, h_in)):
    base = c * 2 * NT + half * NT
    pltpu.sync_copy(hist.at[pl.ds(s * STRIPE, STRIPE)],
                    bnc.at[pl.ds(0, STRIPE)])
    pltpu.sync_copy(bnc.at[pl.ds(0, STRIPE)],
                    out.at[pl.ds(base + s * STRIPE, STRIPE)])


_sc_degrees = pl.kernel(
    _deg_body,
    out_type=jax.ShapeDtypeStruct((NC * 2 * NT,), jnp.float32),
    mesh=_MESH,
    compiler_params=_SC_PARAMS,
    scratch_types=[
        pltpu.VMEM((CH, 128), jnp.int32),
        pltpu.VMEM((CH, 128), jnp.int32),
        pltpu.VMEM((128,), jnp.float32),
        pltpu.VMEM((DSTRIPE,), jnp.float32),
        pltpu.SemaphoreType.DMA,
        pltpu.VMEM_SHARED((NT,), jnp.float32),
        pltpu.VMEM_SHARED((NT,), jnp.float32),
    ],
)


def _agg1_body(x4, src2, dst2, z32, out,
               src_buf, dst_buf, idx_buf, rows, sem, sem2, sem_t, bnc,
               accum):
  c = lax.axis_index("c")
  s = lax.axis_index("s")
  w = c * NS + s
  nblk = 49
  for m, ln in ((0, 1024), (1024, 1024), (2048, 1024), (3072, 56)):
    pltpu.sync_copy(z32.at[pl.ds(0, ln), :], bnc.at[pl.ds(0, ln), :])
    pltpu.sync_copy(bnc.at[pl.ds(0, ln), :],
                    accum.at[pl.ds(s * STRIPE + m, ln), :])
  plsc.subcore_barrier()
  row0 = w * 392

  pltpu.async_copy(src2.at[pl.ds(row0, CH), :], src_buf.at[0], sem_t)
  pltpu.async_copy(dst2.at[pl.ds(row0, CH), :], dst_buf.at[0], sem_t)

  def step(i, carry):
    p = lax.rem(i, 2)
    q = 1 - p
    pltpu.make_async_copy(src2.at[pl.ds(0, CH), :], src_buf.at[p],
                          sem_t).wait()
    pltpu.make_async_copy(dst2.at[pl.ds(0, CH), :], dst_buf.at[p],
                          sem_t).wait()
    off_n = row0 + jnp.minimum(i + 1, nblk - 1) * CH
    pltpu.async_copy(src2.at[pl.ds(off_n, CH), :], src_buf.at[q], sem_t)
    pltpu.async_copy(dst2.at[pl.ds(off_n, CH), :], dst_buf.at[q], sem_t)
    for j in range(CH):
      for v in range(8):
        idx_buf[j, pl.ds(v * 16, 16)] = src_buf[p, j, pl.ds(v * 16, 16)] * 8
    gathers = [
        pltpu.async_copy(x4.at[idx_buf.at[j]], rows.at[j], sem)
        for j in range(CH)]
    scatters = []
    for j in range(CH):
      gathers[j].wait()
      scatters.append(
          pltpu.async_copy(rows.at[j], accum.at[dst_buf.at[p, j]],
                           sem2, add=True))
    for sc in scatters:
      sc.wait()
    return carry

  lax.fori_loop(0, nblk, step, 0)
  pltpu.make_async_copy(src2.at[pl.ds(0, CH), :], src_buf.at[0],
                        sem_t).wait()
  pltpu.make_async_copy(dst2.at[pl.ds(0, CH), :], dst_buf.at[0],
                        sem_t).wait()
  plsc.subcore_barrier()
  for m, ln in ((0, 1024), (1024, 1024), (2048, 1024), (3072, 56)):
    pltpu.sync_copy(accum.at[pl.ds(s * STRIPE + m, ln), :],
                    bnc.at[pl.ds(0, ln), :])
    pltpu.sync_copy(bnc.at[pl.ds(0, ln), :],
                    out.at[c, pl.ds(s * STRIPE + m, ln), pl.ds(0, SL)])


_sc_agg1 = pl.kernel(
    _agg1_body,
    out_type=jax.ShapeDtypeStruct((NC, NT, 128), jnp.float32),
    mesh=_MESH,
    compiler_params=_SC_PARAMS,
    scratch_types=[
        pltpu.VMEM((2, CH, 128), jnp.int32),
        pltpu.VMEM((2, CH, 128), jnp.int32),
        pltpu.VMEM((CH, 128), jnp.int32),
        pltpu.VMEM((CH, 128, SL), jnp.float32),
        pltpu.SemaphoreType.DMA,
        pltpu.SemaphoreType.DMA,
        pltpu.SemaphoreType.DMA,
        pltpu.VMEM((1024, SL), jnp.float32),
        pltpu.VMEM_SHARED((NT, SL), jnp.float32),
    ],
)


def _agg8_body(x8, src2, dst2, z32, out,
               src_buf, dst_buf, idx_buf, rows, sem, sem2, sem_t, bnc,
               accum):
  c = lax.axis_index("c")
  s = lax.axis_index("s")
  nblk = 98
  for k in range(NSL // NC):
    sid = c * (NSL // NC) + k
    for m, ln in ((0, 1024), (1024, 1024), (2048, 1024), (3072, 56)):
      pltpu.sync_copy(z32.at[pl.ds(0, ln), :], bnc.at[pl.ds(0, ln), :])
      pltpu.sync_copy(bnc.at[pl.ds(0, ln), :],
                      accum.at[pl.ds(s * STRIPE + m, ln), :])
    plsc.subcore_barrier()
    row0 = s * 784

    pltpu.async_copy(src2.at[pl.ds(row0, CH), :], src_buf.at[0], sem_t)
    pltpu.async_copy(dst2.at[pl.ds(row0, CH), :], dst_buf.at[0], sem_t)

    def step(i, carry):
      p = lax.rem(i, 2)
      q = 1 - p
      pltpu.make_async_copy(src2.at[pl.ds(0, CH), :], src_buf.at[p],
                            sem_t).wait()
      pltpu.make_async_copy(dst2.at[pl.ds(0, CH), :], dst_buf.at[p],
                            sem_t).wait()
      off_n = row0 + jnp.minimum(i + 1, nblk - 1) * CH
      pltpu.async_copy(src2.at[pl.ds(off_n, CH), :], src_buf.at[q], sem_t)
      pltpu.async_copy(dst2.at[pl.ds(off_n, CH), :], dst_buf.at[q], sem_t)
      for j in range(CH):
        for v in range(8):
          idx_buf[j, pl.ds(v * 16, 16)] = (
              src_buf[p, j, pl.ds(v * 16, 16)] * 16 + sid)
      gathers = [
          pltpu.async_copy(x8.at[idx_buf.at[j]], rows.at[p, j], sem)
          for j in range(CH)]

      @pl.when(i > 0)
      def _():
        for j in range(CH):
          pltpu.make_async_copy(x8.at[idx_buf.at[j]], rows.at[q, j],
                                sem2).wait()

      for j in range(CH):
        gathers[j].wait()
        pltpu.async_copy(rows.at[p, j], accum.at[dst_buf.at[p, j]],
                         sem2, add=True)
      return carry

    lax.fori_loop(0, nblk, step, 0)
    for j in range(CH):
      pltpu.make_async_copy(x8.at[idx_buf.at[j]], rows.at[0, j],
                            sem2).wait()
    pltpu.make_async_copy(src2.at[pl.ds(0, CH), :], src_buf.at[0],
                          sem_t).wait()
    pltpu.make_async_copy(dst2.at[pl.ds(0, CH), :], dst_buf.at[0],
                          sem_t).wait()
    plsc.subcore_barrier()
    for m, ln in ((0, 1024), (1024, 1024), (2048, 1024), (3072, 56)):
      pltpu.sync_copy(accum.at[pl.ds(s * STRIPE + m, ln), :],
                      bnc.at[pl.ds(0, ln), :])
      pltpu.sync_copy(bnc.at[pl.ds(0, ln), :],
                      out.at[sid, pl.ds(s * STRIPE + m, ln), :])
    plsc.subcore_barrier()


_sc_agg8 = pl.kernel(
    _agg8_body,
    out_type=jax.ShapeDtypeStruct((NSL, NT, SL), jnp.float32),
    mesh=_MESH,
    compiler_params=_SC_PARAMS,
    scratch_types=[
        pltpu.VMEM((2, CH, 128), jnp.int32),
        pltpu.VMEM((2, CH, 128), jnp.int32),
        pltpu.VMEM((CH, 128), jnp.int32),
        pltpu.VMEM((2, CH, 128, SL), jnp.float32),
        pltpu.SemaphoreType.DMA,
        pltpu.SemaphoreType.DMA,
        pltpu.SemaphoreType.DMA,
        pltpu.VMEM((1024, SL), jnp.float32),
        pltpu.VMEM_SHARED((NT, SL), jnp.float32),
    ],
)


def _t0_body(degp_ref, a_ref, b_ref):
  d = degp_ref[...]
  a_ref[...] = lax.rsqrt(jnp.maximum(d[0, 0] + d[1, 0], 1.0))
  b_ref[...] = lax.rsqrt(jnp.maximum(d[0, 1] + d[1, 1], 1.0))


def _t1_body(x_ref, a_ref, o_ref):
  o_ref[...] = x_ref[...] * a_ref[...]


def _t2_body(aggp_ref, b_ref, a_ref, w1_ref, b1_ref, wg1_ref,
             h1_ref, y1_ref):
  p = aggp_ref[0, :, :SL] + aggp_ref[1, :, :SL]
  xagg = p * b_ref[...]
  z = jnp.dot(xagg, w1_ref[...], preferred_element_type=jnp.float32)
  h1 = jnp.maximum(z + b1_ref[...], 0.0)
  h1_ref[...] = h1
  y = jnp.dot(h1, wg1_ref[...], preferred_element_type=jnp.float32)
  y1_ref[...] = y * a_ref[...]


def _t3_body(agg_ref, hp_ref, b_ref, a_ref, bg_ref, wg_ref,
             h_ref, y_ref):
  h = jnp.maximum(agg_ref[...] * b_ref[...] + bg_ref[...], 0.0) + hp_ref[...]
  h_ref[...] = h
  y = jnp.dot(h, wg_ref[...], preferred_element_type=jnp.float32)
  y_ref[...] = y * a_ref[...]


def _t4_body(agg_ref, hp_ref, b_ref, bg_ref, wout_ref, bout_ref,
             wc1_ref, bc1_ref, wc2_ref, bc2_ref, wc3_ref, bc3_ref,
             wc4_ref, bc4_ref, out_ref, acc_ref, *, bs):
  i = pl.program_id(0)
  h3 = jnp.maximum(agg_ref[...] * b_ref[...] + bg_ref[...], 0.0) + hp_ref[...]
  feat = jnp.dot(h3, wout_ref[...], preferred_element_type=jnp.float32)
  feat = jnp.maximum(feat + bout_ref[...], 0.0)
  rid = lax.broadcasted_iota(jnp.int32, (bs, 1), 0) + i * bs
  feat = jnp.where(rid < N, feat, 0.0)
  psum = jnp.sum(feat, axis=0, keepdims=True)

  @pl.when(i == 0)
  def _():
    acc_ref[...] = psum

  @pl.when(i > 0)
  def _():
    acc_ref[...] = acc_ref[...] + psum

  @pl.when(i == NT // bs - 1)
  def _():
    pooled = acc_ref[...] * (1.0 / N)
    z = jnp.maximum(
        jnp.dot(pooled, wc1_ref[...], preferred_element_type=jnp.float32)
        + bc1_ref[...], 0.0)
    z = jnp.maximum(
        jnp.dot(z, wc2_ref[...], preferred_element_type=jnp.float32)
        + bc2_ref[...], 0.0)
    z = jnp.maximum(
        jnp.dot(z, wc3_ref[...], preferred_element_type=jnp.float32)
        + bc3_ref[...], 0.0)
    o = jnp.dot(z, wc4_ref[...], preferred_element_type=jnp.float32)
    out_ref[...] = jnp.broadcast_to(o + bc4_ref[...], (8, 128))


def _full(shape):
  return pl.BlockSpec(shape, lambda i: tuple(0 for _ in shape))


def kernel(nodes, edges, W1, b1, Wg1, bg1, Wg2, bg2, Wout, bout,
           Wc1, bc1, Wc2, bc2, Wc3, bc3, Wc4, bc4):
  f32 = jnp.float32
  src = edges[0].astype(jnp.int32).reshape(NC * NS, E // (NC * NS))
  dst = edges[1].astype(jnp.int32).reshape(NC * NS, E // (NC * NS))
  src2 = jnp.pad(src, ((0, 0), (0, 176)),
                 constant_values=DUMMY).reshape(EP_ROWS, 128)
  dst2 = jnp.pad(dst, ((0, 0), (0, 176)),
                 constant_values=DUMMY).reshape(EP_ROWS, 128)
  zdeg = jnp.zeros((DSTRIPE,), f32)
  z32 = jnp.zeros((STRIPE, SL), f32)

  degp = _sc_degrees(src2, dst2, zdeg)
  a2d, b2d = pl.pallas_call(
      _t0_body,
      out_shape=(jax.ShapeDtypeStruct((391, 128), f32),
                 jax.ShapeDtypeStruct((391, 128), f32)),
  )(degp.reshape(NC, 2, 391, 128))
  a = a2d.reshape(NT, 1)
  b = b2d.reshape(NT, 1)

  nodes128 = jnp.pad(nodes, ((0, NT - N), (0, 128 - nodes.shape[1])))
  bs = 3128
  grid = (NT // bs,)
  xs0 = pl.pallas_call(
      _t1_body,
      grid=grid,
      in_specs=[pl.BlockSpec((bs, 128), lambda i: (i, 0)),
                pl.BlockSpec((bs, 1), lambda i: (i, 0))],
      out_specs=pl.BlockSpec((bs, 128), lambda i: (i, 0)),
      out_shape=jax.ShapeDtypeStruct((NT, 128), f32),
  )(nodes128, a)
  aggp0 = _sc_agg1(xs0.reshape(8 * NT, SL), src2, dst2, z32)

  W1p = jnp.pad(W1, ((0, SL - W1.shape[0]), (0, 0)))
  h1, y1 = pl.pallas_call(
      _t2_body,
      grid=grid,
      in_specs=[
          pl.BlockSpec((NC, bs, 128), lambda i: (0, i, 0)),
          pl.BlockSpec((bs, 1), lambda i: (i, 0)),
          pl.BlockSpec((bs, 1), lambda i: (i, 0)),
          _full((SL, HID)),
          _full((1, HID)),
          _full((HID, HID)),
      ],
      out_specs=(pl.BlockSpec((bs, HID), lambda i: (i, 0)),
                 pl.BlockSpec((bs, HID), lambda i: (i, 0))),
      out_shape=(jax.ShapeDtypeStruct((NT, HID), f32),
                 jax.ShapeDtypeStruct((NT, HID), f32)),
  )(aggp0, b, a, W1p, b1[None, :], Wg1)

  def mid_layer(y, hprev, bg, Wnext):
    aggr = _sc_agg8(y.reshape(NSL * NT, SL), src2, dst2, z32)
    agg_cat = aggr.transpose(1, 0, 2).reshape(NT, HID)
    return pl.pallas_call(
        _t3_body,
        grid=grid,
        in_specs=[
            pl.BlockSpec((bs, HID), lambda i: (i, 0)),
            pl.BlockSpec((bs, HID), lambda i: (i, 0)),
            pl.BlockSpec((bs, 1), lambda i: (i, 0)),
            pl.BlockSpec((bs, 1), lambda i: (i, 0)),
            _full((1, HID)),
            _full((HID, HID)),
        ],
        out_specs=(pl.BlockSpec((bs, HID), lambda i: (i, 0)),
                   pl.BlockSpec((bs, HID), lambda i: (i, 0))),
        out_shape=(jax.ShapeDtypeStruct((NT, HID), f32),
                   jax.ShapeDtypeStruct((NT, HID), f32)),
    )(agg_cat, hprev, b, a, bg[None, :], Wnext)

  h2, y2 = mid_layer(y1, h1, bg1, Wg2)

  aggr3 = _sc_agg8(y2.reshape(NSL * NT, SL), src2, dst2, z32)
  agg3 = aggr3.transpose(1, 0, 2).reshape(NT, HID)

  bs4 = 3128
  Wc4p = jnp.pad(Wc4, ((0, 0), (0, 128 - Wc4.shape[1])))
  bc4p = jnp.pad(bc4, (0, 128 - bc4.shape[0]))
  out8 = pl.pallas_call(
      functools.partial(_t4_body, bs=bs4),
      grid=(NT // bs4,),
      in_specs=[
          pl.BlockSpec((bs4, HID), lambda i: (i, 0)),
          pl.BlockSpec((bs4, HID), lambda i: (i, 0)),
          pl.BlockSpec((bs4, 1), lambda i: (i, 0)),
          _full((1, HID)),
          _full((HID, 512)),
          _full((1, 512)),
          _full((512, 1024)),
          _full((1, 1024)),
          _full((1024, 512)),
          _full((1, 512)),
          _full((512, 256)),
          _full((1, 256)),
          _full((256, 128)),
          _full((1, 128)),
      ],
      out_specs=pl.BlockSpec((8, 128), lambda i: (0, 0)),
      out_shape=jax.ShapeDtypeStruct((8, 128), f32),
      scratch_shapes=[pltpu.VMEM((1, 512), f32)],
  )(agg3, h2, b, bg2[None, :], Wout, bout[None, :],
    Wc1, bc1[None, :], Wc2, bc2[None, :], Wc3, bc3[None, :],
    Wc4p, bc4p[None, :])

  return out8[0, :5]


# speedup vs baseline: 12.7666x; 1.1063x over previous
import functools

import jax
import jax.numpy as jnp
from jax import lax
from jax.experimental import pallas as pl
from jax.experimental.pallas import tpu as pltpu
from jax.experimental.pallas import tpu_sc as plsc

N = 50000
E = 1600000
NT = 50048
DUMMY = 50040
EP_ROWS = 12544
HID = 256
SL = 16
NSL = HID // SL
NC = 2
NS = 16
CH = 8
STRIPE = NT // NS
DSTRIPE = NT // 8

_MESH = plsc.VectorSubcoreMesh(
    core_axis_name="c", subcore_axis_name="s", num_cores=NC, num_subcores=NS)
_SC_PARAMS = pltpu.CompilerParams(use_tc_tiling_on_sc=False)


def _deg_body(src2, dst2, zdeg, out, src_buf, dst_buf, ones, bnc, dsem,
              h_out, h_in):
  c = lax.axis_index("c")
  s = lax.axis_index("s")
  w = c * NS + s
  for i in range(8):
    ones[pl.ds(i * 16, 16)] = jnp.full((16,), 1.0, jnp.float32)
  pltpu.sync_copy(zdeg, bnc)

  @pl.when(s < 8)
  def _():
    pltpu.sync_copy(bnc, h_out.at[pl.ds(s * DSTRIPE, DSTRIPE)])

  @pl.when(s >= 8)
  def _():
    pltpu.sync_copy(bnc, h_in.at[pl.ds((s - 8) * DSTRIPE, DSTRIPE)])

  plsc.subcore_barrier()
  row0 = w * 392

  def step(i, carry):
    off = row0 + i * CH
    pltpu.sync_copy(src2.at[pl.ds(off, CH), :], src_buf)
    pltpu.sync_copy(dst2.at[pl.ds(off, CH), :], dst_buf)
    scatters = []
    for j in range(CH):
      scatters.append(
          pltpu.async_copy(ones, h_out.at[src_buf.at[j]], dsem, add=True))
      scatters.append(
          pltpu.async_copy(ones, h_in.at[dst_buf.at[j]], dsem, add=True))
    for sc in scatters:
      sc.wait()
    return carry

  lax.fori_loop(0, 49, step, 0)
  plsc.subcore_barrier()
  for half, hist in ((0, h_out), (1, h_in)):
    base = c * 2 * NT + half * NT
    pltpu.sync_copy(hist.at[pl.ds(s * STRIPE, STRIPE)],
                    bnc.at[pl.ds(0, STRIPE)])
    pltpu.sync_copy(bnc.at[pl.ds(0, STRIPE)],
                    out.at[pl.ds(base + s * STRIPE, STRIPE)])


_sc_degrees = pl.kernel(
    _deg_body,
    out_type=jax.ShapeDtypeStruct((NC * 2 * NT,), jnp.float32),
    mesh=_MESH,
    compiler_params=_SC_PARAMS,
    scratch_types=[
        pltpu.VMEM((CH, 128), jnp.int32),
        pltpu.VMEM((CH, 128), jnp.int32),
        pltpu.VMEM((128,), jnp.float32),
        pltpu.VMEM((DSTRIPE,), jnp.float32),
        pltpu.SemaphoreType.DMA,
        pltpu.VMEM_SHARED((NT,), jnp.float32),
        pltpu.VMEM_SHARED((NT,), jnp.float32),
    ],
)


def _agg1_body(x4, src2, dst2, z32, out,
               src_buf, dst_buf, idx_buf, rows, sem, sem2, sem_t, bnc,
               accum):
  c = lax.axis_index("c")
  s = lax.axis_index("s")
  w = c * NS + s
  nblk = 49
  for m, ln in ((0, 1024), (1024, 1024), (2048, 1024), (3072, 56)):
    pltpu.sync_copy(z32.at[pl.ds(0, ln), :], bnc.at[pl.ds(0, ln), :])
    pltpu.sync_copy(bnc.at[pl.ds(0, ln), :],
                    accum.at[pl.ds(s * STRIPE + m, ln), :])
  plsc.subcore_barrier()
  row0 = w * 392

  pltpu.async_copy(src2.at[pl.ds(row0, CH), :], src_buf.at[0], sem_t)
  pltpu.async_copy(dst2.at[pl.ds(row0, CH), :], dst_buf.at[0], sem_t)

  def step(i, carry):
    p = lax.rem(i, 2)
    q = 1 - p
    pltpu.make_async_copy(src2.at[pl.ds(0, CH), :], src_buf.at[p],
                          sem_t).wait()
    pltpu.make_async_copy(dst2.at[pl.ds(0, CH), :], dst_buf.at[p],
                          sem_t).wait()
    off_n = row0 + jnp.minimum(i + 1, nblk - 1) * CH
    pltpu.async_copy(src2.at[pl.ds(off_n, CH), :], src_buf.at[q], sem_t)
    pltpu.async_copy(dst2.at[pl.ds(off_n, CH), :], dst_buf.at[q], sem_t)
    for j in range(CH):
      for v in range(8):
        idx_buf[j, pl.ds(v * 16, 16)] = src_buf[p, j, pl.ds(v * 16, 16)] * 8
    gathers = [
        pltpu.async_copy(x4.at[idx_buf.at[j]], rows.at[j], sem)
        for j in range(CH)]
    scatters = []
    for j in range(CH):
      gathers[j].wait()
      scatters.append(
          pltpu.async_copy(rows.at[j], accum.at[dst_buf.at[p, j]],
                           sem2, add=True))
    for sc in scatters:
      sc.wait()
    return carry

  lax.fori_loop(0, nblk, step, 0)
  pltpu.make_async_copy(src2.at[pl.ds(0, CH), :], src_buf.at[0],
                        sem_t).wait()
  pltpu.make_async_copy(dst2.at[pl.ds(0, CH), :], dst_buf.at[0],
                        sem_t).wait()
  plsc.subcore_barrier()
  for m, ln in ((0, 1024), (1024, 1024), (2048, 1024), (3072, 56)):
    pltpu.sync_copy(accum.at[pl.ds(s * STRIPE + m, ln), :],
                    bnc.at[pl.ds(0, ln), :])
    pltpu.sync_copy(bnc.at[pl.ds(0, ln), :],
                    out.at[c, pl.ds(s * STRIPE + m, ln), pl.ds(0, SL)])


_sc_agg1 = pl.kernel(
    _agg1_body,
    out_type=jax.ShapeDtypeStruct((NC, NT, 128), jnp.float32),
    mesh=_MESH,
    compiler_params=_SC_PARAMS,
    scratch_types=[
        pltpu.VMEM((2, CH, 128), jnp.int32),
        pltpu.VMEM((2, CH, 128), jnp.int32),
        pltpu.VMEM((CH, 128), jnp.int32),
        pltpu.VMEM((CH, 128, SL), jnp.float32),
        pltpu.SemaphoreType.DMA,
        pltpu.SemaphoreType.DMA,
        pltpu.SemaphoreType.DMA,
        pltpu.VMEM((1024, SL), jnp.float32),
        pltpu.VMEM_SHARED((NT, SL), jnp.float32),
    ],
)


def _agg8_body(x8, src2, dst2, z32, out,
               src_buf, dst_buf, idx_buf, rows, sem, sem2, sem_t, bnc,
               accum):
  c = lax.axis_index("c")
  s = lax.axis_index("s")
  nblk = 98
  for k in range(NSL // NC):
    sid = c * (NSL // NC) + k
    for m, ln in ((0, 1024), (1024, 1024), (2048, 1024), (3072, 56)):
      pltpu.sync_copy(z32.at[pl.ds(0, ln), :], bnc.at[pl.ds(0, ln), :])
      pltpu.sync_copy(bnc.at[pl.ds(0, ln), :],
                      accum.at[pl.ds(s * STRIPE + m, ln), :])
    plsc.subcore_barrier()
    row0 = s * 784

    pltpu.async_copy(src2.at[pl.ds(row0, CH), :], src_buf.at[0], sem_t)
    pltpu.async_copy(dst2.at[pl.ds(row0, CH), :], dst_buf.at[0], sem_t)

    def step(i, carry):
      p = lax.rem(i, 2)
      q = 1 - p
      pltpu.make_async_copy(src2.at[pl.ds(0, CH), :], src_buf.at[p],
                            sem_t).wait()
      pltpu.make_async_copy(dst2.at[pl.ds(0, CH), :], dst_buf.at[p],
                            sem_t).wait()
      off_n = row0 + jnp.minimum(i + 1, nblk - 1) * CH
      pltpu.async_copy(src2.at[pl.ds(off_n, CH), :], src_buf.at[q], sem_t)
      pltpu.async_copy(dst2.at[pl.ds(off_n, CH), :], dst_buf.at[q], sem_t)
      for j in range(CH):
        for v in range(8):
          idx_buf[j, pl.ds(v * 16, 16)] = (
              src_buf[p, j, pl.ds(v * 16, 16)] * 16 + sid)
      gathers = [
          pltpu.async_copy(x8.at[idx_buf.at[j]], rows.at[p, j], sem)
          for j in range(CH)]

      @pl.when(i > 0)
      def _():
        for j in range(CH):
          pltpu.make_async_copy(x8.at[idx_buf.at[j]], rows.at[q, j],
                                sem2).wait()

      for j in range(CH):
        gathers[j].wait()
        pltpu.async_copy(rows.at[p, j], accum.at[dst_buf.at[p, j]],
                         sem2, add=True)
      return carry

    lax.fori_loop(0, nblk, step, 0)
    for j in range(CH):
      pltpu.make_async_copy(x8.at[idx_buf.at[j]], rows.at[0, j],
                            sem2).wait()
    pltpu.make_async_copy(src2.at[pl.ds(0, CH), :], src_buf.at[0],
                          sem_t).wait()
    pltpu.make_async_copy(dst2.at[pl.ds(0, CH), :], dst_buf.at[0],
                          sem_t).wait()
    plsc.subcore_barrier()
    for m, ln in ((0, 1024), (1024, 1024), (2048, 1024), (3072, 56)):
      pltpu.sync_copy(accum.at[pl.ds(s * STRIPE + m, ln), :],
                      bnc.at[pl.ds(0, ln), :])
      pltpu.sync_copy(bnc.at[pl.ds(0, ln), :],
                      out.at[pl.ds(s * STRIPE + m, ln), pl.ds(sid * SL, SL)])
    plsc.subcore_barrier()


_sc_agg8 = pl.kernel(
    _agg8_body,
    out_type=jax.ShapeDtypeStruct((NT, HID), jnp.float32),
    mesh=_MESH,
    compiler_params=_SC_PARAMS,
    scratch_types=[
        pltpu.VMEM((2, CH, 128), jnp.int32),
        pltpu.VMEM((2, CH, 128), jnp.int32),
        pltpu.VMEM((CH, 128), jnp.int32),
        pltpu.VMEM((2, CH, 128, SL), jnp.float32),
        pltpu.SemaphoreType.DMA,
        pltpu.SemaphoreType.DMA,
        pltpu.SemaphoreType.DMA,
        pltpu.VMEM((1024, SL), jnp.float32),
        pltpu.VMEM_SHARED((NT, SL), jnp.float32),
    ],
)


def _t0_body(degp_ref, a_ref, b_ref):
  d = degp_ref[...]
  a_ref[...] = lax.rsqrt(jnp.maximum(d[0, 0] + d[1, 0], 1.0))
  b_ref[...] = lax.rsqrt(jnp.maximum(d[0, 1] + d[1, 1], 1.0))


def _t1_body(x_ref, a_ref, o_ref):
  o_ref[...] = x_ref[...] * a_ref[...]


def _t2_body(aggp_ref, b_ref, a_ref, w1_ref, b1_ref, wg1_ref,
             h1_ref, y1_ref):
  p = aggp_ref[0, :, :SL] + aggp_ref[1, :, :SL]
  xagg = p * b_ref[...]
  z = jnp.dot(xagg, w1_ref[...], preferred_element_type=jnp.float32)
  h1 = jnp.maximum(z + b1_ref[...], 0.0)
  h1_ref[...] = h1
  y = jnp.dot(h1, wg1_ref[...], preferred_element_type=jnp.float32)
  y1_ref[...] = y * a_ref[...]


def _t3_body(agg_ref, hp_ref, b_ref, a_ref, bg_ref, wg_ref,
             h_ref, y_ref):
  h = jnp.maximum(agg_ref[...] * b_ref[...] + bg_ref[...], 0.0) + hp_ref[...]
  h_ref[...] = h
  y = jnp.dot(h, wg_ref[...], preferred_element_type=jnp.float32)
  y_ref[...] = y * a_ref[...]


def _t4_body(agg_ref, hp_ref, b_ref, bg_ref, wout_ref, bout_ref,
             wc1_ref, bc1_ref, wc2_ref, bc2_ref, wc3_ref, bc3_ref,
             wc4_ref, bc4_ref, out_ref, acc_ref, *, bs):
  i = pl.program_id(0)
  h3 = jnp.maximum(agg_ref[...] * b_ref[...] + bg_ref[...], 0.0) + hp_ref[...]
  feat = jnp.dot(h3, wout_ref[...], preferred_element_type=jnp.float32)
  feat = jnp.maximum(feat + bout_ref[...], 0.0)
  rid = lax.broadcasted_iota(jnp.int32, (bs, 1), 0) + i * bs
  feat = jnp.where(rid < N, feat, 0.0)
  psum = jnp.sum(feat, axis=0, keepdims=True)

  @pl.when(i == 0)
  def _():
    acc_ref[...] = psum

  @pl.when(i > 0)
  def _():
    acc_ref[...] = acc_ref[...] + psum

  @pl.when(i == NT // bs - 1)
  def _():
    pooled = acc_ref[...] * (1.0 / N)
    z = jnp.maximum(
        jnp.dot(pooled, wc1_ref[...], preferred_element_type=jnp.float32)
        + bc1_ref[...], 0.0)
    z = jnp.maximum(
        jnp.dot(z, wc2_ref[...], preferred_element_type=jnp.float32)
        + bc2_ref[...], 0.0)
    z = jnp.maximum(
        jnp.dot(z, wc3_ref[...], preferred_element_type=jnp.float32)
        + bc3_ref[...], 0.0)
    o = jnp.dot(z, wc4_ref[...], preferred_element_type=jnp.float32)
    out_ref[...] = jnp.broadcast_to(o + bc4_ref[...], (8, 128))


def _full(shape):
  return pl.BlockSpec(shape, lambda i: tuple(0 for _ in shape))


def kernel(nodes, edges, W1, b1, Wg1, bg1, Wg2, bg2, Wout, bout,
           Wc1, bc1, Wc2, bc2, Wc3, bc3, Wc4, bc4):
  f32 = jnp.float32
  src = edges[0].astype(jnp.int32).reshape(NC * NS, E // (NC * NS))
  dst = edges[1].astype(jnp.int32).reshape(NC * NS, E // (NC * NS))
  src2 = jnp.pad(src, ((0, 0), (0, 176)),
                 constant_values=DUMMY).reshape(EP_ROWS, 128)
  dst2 = jnp.pad(dst, ((0, 0), (0, 176)),
                 constant_values=DUMMY).reshape(EP_ROWS, 128)
  zdeg = jnp.zeros((DSTRIPE,), f32)
  z32 = jnp.zeros((STRIPE, SL), f32)

  degp = _sc_degrees(src2, dst2, zdeg)
  a2d, b2d = pl.pallas_call(
      _t0_body,
      out_shape=(jax.ShapeDtypeStruct((391, 128), f32),
                 jax.ShapeDtypeStruct((391, 128), f32)),
  )(degp.reshape(NC, 2, 391, 128))
  a = a2d.reshape(NT, 1)
  b = b2d.reshape(NT, 1)

  nodes128 = jnp.pad(nodes, ((0, NT - N), (0, 128 - nodes.shape[1])))
  bs = 3128
  grid = (NT // bs,)
  xs0 = pl.pallas_call(
      _t1_body,
      grid=grid,
      in_specs=[pl.BlockSpec((bs, 128), lambda i: (i, 0)),
                pl.BlockSpec((bs, 1), lambda i: (i, 0))],
      out_specs=pl.BlockSpec((bs, 128), lambda i: (i, 0)),
      out_shape=jax.ShapeDtypeStruct((NT, 128), f32),
  )(nodes128, a)
  aggp0 = _sc_agg1(xs0.reshape(8 * NT, SL), src2, dst2, z32)

  W1p = jnp.pad(W1, ((0, SL - W1.shape[0]), (0, 0)))
  h1, y1 = pl.pallas_call(
      _t2_body,
      grid=grid,
      in_specs=[
          pl.BlockSpec((NC, bs, 128), lambda i: (0, i, 0)),
          pl.BlockSpec((bs, 1), lambda i: (i, 0)),
          pl.BlockSpec((bs, 1), lambda i: (i, 0)),
          _full((SL, HID)),
          _full((1, HID)),
          _full((HID, HID)),
      ],
      out_specs=(pl.BlockSpec((bs, HID), lambda i: (i, 0)),
                 pl.BlockSpec((bs, HID), lambda i: (i, 0))),
      out_shape=(jax.ShapeDtypeStruct((NT, HID), f32),
                 jax.ShapeDtypeStruct((NT, HID), f32)),
  )(aggp0, b, a, W1p, b1[None, :], Wg1)

  def mid_layer(y, hprev, bg, Wnext):
    agg_cat = _sc_agg8(y.reshape(NSL * NT, SL), src2, dst2, z32)
    return pl.pallas_call(
        _t3_body,
        grid=grid,
        in_specs=[
            pl.BlockSpec((bs, HID), lambda i: (i, 0)),
            pl.BlockSpec((bs, HID), lambda i: (i, 0)),
            pl.BlockSpec((bs, 1), lambda i: (i, 0)),
            pl.BlockSpec((bs, 1), lambda i: (i, 0)),
            _full((1, HID)),
            _full((HID, HID)),
        ],
        out_specs=(pl.BlockSpec((bs, HID), lambda i: (i, 0)),
                   pl.BlockSpec((bs, HID), lambda i: (i, 0))),
        out_shape=(jax.ShapeDtypeStruct((NT, HID), f32),
                   jax.ShapeDtypeStruct((NT, HID), f32)),
    )(agg_cat, hprev, b, a, bg[None, :], Wnext)

  h2, y2 = mid_layer(y1, h1, bg1, Wg2)

  agg3 = _sc_agg8(y2.reshape(NSL * NT, SL), src2, dst2, z32)

  bs4 = 3128
  Wc4p = jnp.pad(Wc4, ((0, 0), (0, 128 - Wc4.shape[1])))
  bc4p = jnp.pad(bc4, (0, 128 - bc4.shape[0]))
  out8 = pl.pallas_call(
      functools.partial(_t4_body, bs=bs4),
      grid=(NT // bs4,),
      in_specs=[
          pl.BlockSpec((bs4, HID), lambda i: (i, 0)),
          pl.BlockSpec((bs4, HID), lambda i: (i, 0)),
          pl.BlockSpec((bs4, 1), lambda i: (i, 0)),
          _full((1, HID)),
          _full((HID, 512)),
          _full((1, 512)),
          _full((512, 1024)),
          _full((1, 1024)),
          _full((1024, 512)),
          _full((1, 512)),
          _full((512, 256)),
          _full((1, 256)),
          _full((256, 128)),
          _full((1, 128)),
      ],
      out_specs=pl.BlockSpec((8, 128), lambda i: (0, 0)),
      out_shape=jax.ShapeDtypeStruct((8, 128), f32),
      scratch_shapes=[pltpu.VMEM((1, 512), f32)],
  )(agg3, h2, b, bg2[None, :], Wout, bout[None, :],
    Wc1, bc1[None, :], Wc2, bc2[None, :], Wc3, bc3[None, :],
    Wc4p, bc4p[None, :])

  return out8[0, :5]
